# Initial kernel scaffold; baseline (speedup 1.0000x reference)
import jax
import jax.numpy as jnp
from jax.experimental import pallas as pl


def kernel(feats, weights, table, bias):
    raise NotImplementedError("write your pallas kernel here")


# speedup vs baseline: 2.5067x; 2.5067x over previous
import dataclasses
import functools

import jax
import jax.numpy as jnp
from jax import lax
from jax.experimental import pallas as pl
from jax.experimental.pallas import tpu as pltpu
from jax.experimental.pallas import tpu_sc as plsc

LANES = 16
NC, NS = 2, 16
NW = NC * NS


def kernel(feats, weights, table, bias):
    B, L = feats.shape
    V, D = table.shape
    KD = D // LANES
    RPW = B // NW
    CB = 8
    NSTEPS = RPW // CB
    IDX = CB * L
    IPG = 2 * L
    NG = IDX // IPG

    feats_r = feats.astype(jnp.int32).reshape(NW, NSTEPS, NG, IPG)
    w_r = weights.reshape(NW, NSTEPS, IDX)

    mesh = plsc.VectorSubcoreMesh(core_axis_name="c", subcore_axis_name="s")

    cp = pltpu.CompilerParams()
    if "needs_layout_passes" in pltpu.CompilerParams.__dataclass_fields__:
        cp = dataclasses.replace(cp, needs_layout_passes=False)
    if "use_tc_tiling_on_sc" in pltpu.CompilerParams.__dataclass_fields__:
        cp = dataclasses.replace(cp, use_tc_tiling_on_sc=False)

    @functools.partial(
        pl.kernel,
        compiler_params=cp,
        out_type=jax.ShapeDtypeStruct((B, D), jnp.float32),
        mesh=mesh,
        scratch_types=[
            pltpu.VMEM((2, NG, IPG), jnp.int32),
            pltpu.VMEM((IDX,), jnp.float32),
            pltpu.VMEM((IDX,), jnp.float32),
            pltpu.VMEM((IDX, D), jnp.float32),
            pltpu.VMEM((IDX, D), jnp.float32),
            pltpu.VMEM((CB, D), jnp.float32),
            pltpu.VMEM((D,), jnp.float32),
            pltpu.SemaphoreType.DMA,
            pltpu.SemaphoreType.DMA,
        ],
    )
    def run(feats_hbm, w_hbm, table_hbm, bias_hbm, out_hbm,
            idx_v, w_v0, w_v1, rows_v0, rows_v1, out_v, bias_v,
            sem_g0, sem_g1):
        wid = lax.axis_index("s") * NC + lax.axis_index("c")
        gsems = (sem_g0, sem_g1)
        wbufs = (w_v0, w_v1)
        rbufs = (rows_v0, rows_v1)

        pltpu.sync_copy(bias_hbm, bias_v)

        def load_step(s, buf):
            pltpu.sync_copy(feats_hbm.at[wid, s], idx_v.at[buf])
            pltpu.sync_copy(w_hbm.at[wid, s], wbufs[buf])
            for g in range(NG):
                pltpu.async_copy(
                    table_hbm.at[idx_v.at[buf, g]],
                    rbufs[buf].at[pl.ds(g * IPG, IPG)],
                    gsems[buf])

        def wait_step(buf):
            for g in range(NG):
                pltpu.make_async_copy(
                    table_hbm.at[idx_v.at[buf, g]],
                    rbufs[buf].at[pl.ds(g * IPG, IPG)],
                    gsems[buf]).wait()

        def compute_step(s, buf):
            rows = rbufs[buf]
            wref = wbufs[buf]

            @pl.loop(0, CB)
            def _(b):
                base = b * L
                acc0 = tuple(bias_v[pl.ds(k * LANES, LANES)] for k in range(KD))

                def lbody(l, accs):
                    i = base + l
                    wspl = plsc.load_gather(
                        wref, [jnp.zeros((LANES,), jnp.int32) + i])
                    return tuple(
                        accs[k] + wspl * rows[i, pl.ds(k * LANES, LANES)]
                        for k in range(KD))

                accs = lax.fori_loop(0, L, lbody, acc0)
                for k in range(KD):
                    out_v[b, pl.ds(k * LANES, LANES)] = accs[k]

            pltpu.sync_copy(out_v, out_hbm.at[pl.ds(wid * RPW + s * CB, CB)])

        load_step(0, 0)

        @pl.loop(0, NSTEPS // 2)
        def _(it):
            for half in range(2):
                s = it * 2 + half
                buf = half

                @pl.when(s + 1 < NSTEPS)
                def _():
                    load_step(s + 1, 1 - buf)

                wait_step(buf)
                compute_step(s, buf)

    return run(feats_r, w_r, table, bias)


# speedup vs baseline: 2.5864x; 1.0318x over previous
import dataclasses
import functools

import jax
import jax.numpy as jnp
from jax import lax
from jax.experimental import pallas as pl
from jax.experimental.pallas import tpu as pltpu
from jax.experimental.pallas import tpu_sc as plsc

LANES = 16
NC, NS = 2, 16
NW = NC * NS


def kernel(feats, weights, table, bias):
    B, L = feats.shape
    V, D = table.shape
    KD = D // LANES
    RPW = B // NW
    CB = 8
    NSTEPS = RPW // CB
    IDX = CB * L
    IPG = 2 * L
    NG = IDX // IPG

    feats_r = feats.astype(jnp.int32).reshape(NW, NSTEPS, NG, IPG)
    w_r = weights.reshape(NW, NSTEPS, IDX)

    mesh = plsc.VectorSubcoreMesh(core_axis_name="c", subcore_axis_name="s")

    cp = pltpu.CompilerParams()
    if "needs_layout_passes" in pltpu.CompilerParams.__dataclass_fields__:
        cp = dataclasses.replace(cp, needs_layout_passes=False)
    if "use_tc_tiling_on_sc" in pltpu.CompilerParams.__dataclass_fields__:
        cp = dataclasses.replace(cp, use_tc_tiling_on_sc=False)

    @functools.partial(
        pl.kernel,
        compiler_params=cp,
        out_type=jax.ShapeDtypeStruct((B, D), jnp.float32),
        mesh=mesh,
        scratch_types=[
            pltpu.VMEM((2, NG, IPG), jnp.int32),
            pltpu.VMEM((IDX,), jnp.float32),
            pltpu.VMEM((IDX,), jnp.float32),
            pltpu.VMEM((IDX, D), jnp.float32),
            pltpu.VMEM((IDX, D), jnp.float32),
            pltpu.VMEM((CB, D), jnp.float32),
            pltpu.VMEM((D,), jnp.float32),
            pltpu.SemaphoreType.DMA,
            pltpu.SemaphoreType.DMA,
        ],
    )
    def run(feats_hbm, w_hbm, table_hbm, bias_hbm, out_hbm,
            idx_v, w_v0, w_v1, rows_v0, rows_v1, out_v, bias_v,
            sem_g0, sem_g1):
        wid = lax.axis_index("s") * NC + lax.axis_index("c")
        gsems = (sem_g0, sem_g1)
        wbufs = (w_v0, w_v1)
        rbufs = (rows_v0, rows_v1)

        pltpu.sync_copy(bias_hbm, bias_v)
        bchunks = tuple(bias_v[pl.ds(k * LANES, LANES)] for k in range(KD))

        def lane_bcast(vec, lane):
            dn = lax.GatherDimensionNumbers(
                offset_dims=(), collapsed_slice_dims=(0,), start_index_map=(0,))
            idx = jnp.full((LANES, 1), lane, jnp.int32)
            return lax.gather(vec, idx, dn, slice_sizes=(1,),
                              mode=lax.GatherScatterMode.PROMISE_IN_BOUNDS)

        def load_step(s, buf):
            pltpu.sync_copy(feats_hbm.at[wid, s], idx_v.at[buf])
            pltpu.sync_copy(w_hbm.at[wid, s], wbufs[buf])
            for g in range(NG):
                pltpu.async_copy(
                    table_hbm.at[idx_v.at[buf, g]],
                    rbufs[buf].at[pl.ds(g * IPG, IPG)],
                    gsems[buf])

        def wait_step(buf):
            for g in range(NG):
                pltpu.make_async_copy(
                    table_hbm.at[idx_v.at[buf, g]],
                    rbufs[buf].at[pl.ds(g * IPG, IPG)],
                    gsems[buf]).wait()

        def compute_step(s, buf):
            rows = rbufs[buf]
            wref = wbufs[buf]

            @pl.loop(0, CB)
            def _(b):
                base = b * L
                wv = (wref[pl.ds(base, LANES)],
                      wref[pl.ds(base + 16, LANES)],
                      wref[pl.ds(base + 32, LANES)],
                      wref[pl.ds(base + 34, LANES)])
                acc_e = list(bchunks)
                acc_o = [jnp.zeros((LANES,), jnp.float32) for _ in range(KD)]
                for l in range(L):
                    if l < 48:
                        src, lane = divmod(l, 16)
                    else:
                        src, lane = 3, l - 34
                    wspl = lane_bcast(wv[src], lane)
                    tgt = acc_e if l % 2 == 0 else acc_o
                    for k in range(KD):
                        tgt[k] = tgt[k] + wspl * rows[base + l,
                                                      pl.ds(k * LANES, LANES)]
                for k in range(KD):
                    out_v[b, pl.ds(k * LANES, LANES)] = acc_e[k] + acc_o[k]

            pltpu.sync_copy(out_v, out_hbm.at[pl.ds(wid * RPW + s * CB, CB)])

        load_step(0, 0)

        @pl.loop(0, NSTEPS // 2)
        def _(it):
            for half in range(2):
                s = it * 2 + half
                buf = half

                @pl.when(s + 1 < NSTEPS)
                def _():
                    load_step(s + 1, 1 - buf)

                wait_step(buf)
                compute_step(s, buf)

    return run(feats_r, w_r, table, bias)
